# baseline (speedup 1.0000x reference)
import jax
import jax.numpy as jnp
from jax import lax
from jax.experimental import pallas as pl
from jax.experimental.pallas import tpu as pltpu


def _rvq_body(x_ref, cbt_ref, split_ref, cn2_ref,
              out_ref, idx_ref, loss_ref):
    M = x_ref.shape[0]
    Q = cbt_ref.shape[0]
    K = cbt_ref.shape[2]

    r = x_ref[...]
    out = jnp.zeros_like(r)
    iota_k = lax.broadcasted_iota(jnp.int32, (M, K), 1)
    lane128 = lax.broadcasted_iota(jnp.int32, (M, 128), 1)
    lane8 = lax.broadcasted_iota(jnp.int32, (8, 128), 1)
    sub8 = lax.broadcasted_iota(jnp.int32, (8, 128), 0)
    idxacc = jnp.zeros((M, 128), dtype=jnp.int32)
    lvec = jnp.zeros((8, 128), dtype=jnp.float32)

    DIM = x_ref.shape[1]
    for q in range(Q):
        e2 = jnp.dot(r * -2.0, cbt_ref[q], preferred_element_type=jnp.float32)
        rn2 = jnp.sum(r * r, axis=1, keepdims=True)
        d = (rn2 + e2) + cn2_ref[q]
        dmin = jnp.min(d, axis=1, keepdims=True)
        idx = jnp.min(jnp.where(d == dmin, iota_k, K), axis=1, keepdims=True)
        onehot = (iota_k == idx).astype(jnp.bfloat16)
        qcat = jnp.dot(onehot, split_ref[q], preferred_element_type=jnp.float32)
        quant = ((qcat[:, :DIM] + qcat[:, DIM:2 * DIM])
                 + qcat[:, 2 * DIM:])
        diff = quant - r
        lvec = lvec + jnp.where((lane8 == q) & (sub8 == 0),
                                jnp.sum(diff * diff), 0.0)
        idxacc = jnp.where(lane128 == q, jnp.broadcast_to(idx, (M, 128)), idxacc)
        out = out + (r + diff)
        r = r - quant

    out_ref[...] = out
    idx_ref[...] = idxacc
    loss_ref[0] = lvec


def _rvq_shard(x, codebooks, total_elems):
    B, N, DIM = x.shape
    Q, K, _ = codebooks.shape
    BN = B * N
    M = 2304 if BN % 2304 == 0 else BN
    G = BN // M

    xf = x.reshape(BN, DIM)
    cbt = jnp.transpose(codebooks, (0, 2, 1))
    cn2 = jnp.stack([jnp.sum(codebooks[q] * codebooks[q], axis=-1)
                     for q in range(Q)])[:, None, :]
    def _trunc_bf16(v):
        bits = lax.bitcast_convert_type(v, jnp.uint32)
        return lax.bitcast_convert_type(bits & jnp.uint32(0xFFFF0000),
                                        jnp.float32)
    hi_f = _trunc_bf16(codebooks)
    rem1 = codebooks - hi_f
    mid_f = _trunc_bf16(rem1)
    rem2 = rem1 - mid_f
    split = jnp.concatenate([hi_f.astype(jnp.bfloat16),
                             mid_f.astype(jnp.bfloat16),
                             rem2.astype(jnp.bfloat16)], axis=-1)

    out, idx_raw, loss_raw = pl.pallas_call(
        _rvq_body,
        grid=(G,),
        in_specs=[
            pl.BlockSpec((M, DIM), lambda i: (i, 0)),
            pl.BlockSpec((Q, DIM, K), lambda i: (0, 0, 0)),
            pl.BlockSpec((Q, K, 3 * DIM), lambda i: (0, 0, 0)),
            pl.BlockSpec((Q, 1, K), lambda i: (0, 0, 0)),
        ],
        out_specs=[
            pl.BlockSpec((M, DIM), lambda i: (i, 0)),
            pl.BlockSpec((M, 128), lambda i: (i, 0)),
            pl.BlockSpec((1, 8, 128), lambda i: (i, 0, 0)),
        ],
        out_shape=[
            jax.ShapeDtypeStruct((BN, DIM), jnp.float32),
            jax.ShapeDtypeStruct((BN, 128), jnp.int32),
            jax.ShapeDtypeStruct((G, 8, 128), jnp.float32),
        ],
        compiler_params=pltpu.CompilerParams(
            dimension_semantics=("arbitrary",),
        ),
    )(xf, cbt, split, cn2)

    quantized_out = out.reshape(B, N, DIM)
    indices = idx_raw[:, :Q].reshape(B, N, Q)
    losses = loss_raw.sum(axis=(0, 1))[:Q] / total_elems
    return quantized_out, indices, losses


def kernel(x, codebooks):
    B, N, DIM = x.shape
    return _rvq_shard(x, codebooks, B * N * DIM)


# speedup vs baseline: 1.2436x; 1.2436x over previous
import jax
import jax.numpy as jnp
from jax import lax
from jax.experimental import pallas as pl
from jax.experimental.pallas import tpu as pltpu


def _rvq_body(x_ref, cbt_ref, split_ref, cn2_ref,
              out_ref, idx_ref, loss_ref):
    M = x_ref.shape[0]
    Q = cbt_ref.shape[0]
    K = cbt_ref.shape[2]

    r = x_ref[...]
    out = jnp.zeros_like(r)
    iota_k = lax.broadcasted_iota(jnp.int32, (M, K), 1)
    lane128 = lax.broadcasted_iota(jnp.int32, (M, 128), 1)
    lane8 = lax.broadcasted_iota(jnp.int32, (8, 128), 1)
    sub8 = lax.broadcasted_iota(jnp.int32, (8, 128), 0)
    idxacc = jnp.zeros((M, 128), dtype=jnp.int32)
    lvec = jnp.zeros((8, 128), dtype=jnp.float32)

    DIM = x_ref.shape[1]
    for q in range(Q):
        e2 = jnp.dot(r * -2.0, cbt_ref[q], preferred_element_type=jnp.float32)
        rn2 = jnp.sum(r * r, axis=1, keepdims=True)
        d = (rn2 + e2) + cn2_ref[q]
        dmin = jnp.min(d, axis=1, keepdims=True)
        idx = jnp.min(jnp.where(d == dmin, iota_k, K), axis=1, keepdims=True)
        onehot = (iota_k == idx).astype(jnp.bfloat16)
        qcat = jnp.dot(onehot, split_ref[q], preferred_element_type=jnp.float32)
        quant = ((qcat[:, :DIM] + qcat[:, DIM:2 * DIM])
                 + qcat[:, 2 * DIM:])
        diff = quant - r
        lvec = lvec + jnp.where((lane8 == q) & (sub8 == 0),
                                jnp.sum(diff * diff), 0.0)
        idxacc = jnp.where(lane128 == q, jnp.broadcast_to(idx, (M, 128)), idxacc)
        out = out + (r + diff)
        r = r - quant

    out_ref[...] = out
    idx_ref[...] = idxacc
    loss_ref[0] = lvec


def _rvq_shard(x, codebooks, total_elems):
    B, N, DIM = x.shape
    Q, K, _ = codebooks.shape
    BN = B * N
    M = 576 if BN % 576 == 0 else BN
    G = BN // M

    xf = x.reshape(BN, DIM)
    cbt = jnp.transpose(codebooks, (0, 2, 1))
    cn2 = jnp.stack([jnp.sum(codebooks[q] * codebooks[q], axis=-1)
                     for q in range(Q)])[:, None, :]
    def _trunc_bf16(v):
        bits = lax.bitcast_convert_type(v, jnp.uint32)
        return lax.bitcast_convert_type(bits & jnp.uint32(0xFFFF0000),
                                        jnp.float32)
    hi_f = _trunc_bf16(codebooks)
    rem1 = codebooks - hi_f
    mid_f = _trunc_bf16(rem1)
    rem2 = rem1 - mid_f
    split = jnp.concatenate([hi_f.astype(jnp.bfloat16),
                             mid_f.astype(jnp.bfloat16),
                             rem2.astype(jnp.bfloat16)], axis=-1)

    out, idx_raw, loss_raw = pl.pallas_call(
        _rvq_body,
        grid=(G,),
        in_specs=[
            pl.BlockSpec((M, DIM), lambda i: (i, 0)),
            pl.BlockSpec((Q, DIM, K), lambda i: (0, 0, 0)),
            pl.BlockSpec((Q, K, 3 * DIM), lambda i: (0, 0, 0)),
            pl.BlockSpec((Q, 1, K), lambda i: (0, 0, 0)),
        ],
        out_specs=[
            pl.BlockSpec((M, DIM), lambda i: (i, 0)),
            pl.BlockSpec((M, 128), lambda i: (i, 0)),
            pl.BlockSpec((1, 8, 128), lambda i: (i, 0, 0)),
        ],
        out_shape=[
            jax.ShapeDtypeStruct((BN, DIM), jnp.float32),
            jax.ShapeDtypeStruct((BN, 128), jnp.int32),
            jax.ShapeDtypeStruct((G, 8, 128), jnp.float32),
        ],
        compiler_params=pltpu.CompilerParams(
            dimension_semantics=("arbitrary",),
        ),
    )(xf, cbt, split, cn2)

    quantized_out = out.reshape(B, N, DIM)
    indices = idx_raw[:, :Q].reshape(B, N, Q)
    losses = loss_raw.sum(axis=(0, 1))[:Q] / total_elems
    return quantized_out, indices, losses


def kernel(x, codebooks):
    B, N, DIM = x.shape
    return _rvq_shard(x, codebooks, B * N * DIM)


# speedup vs baseline: 1.3032x; 1.0479x over previous
import jax
import jax.numpy as jnp
from jax import lax
from jax.experimental import pallas as pl
from jax.experimental.pallas import tpu as pltpu


def _rvq_body(x_ref, cbt_ref, split_ref, cn2_ref,
              out_ref, idx_ref, loss_ref):
    M = x_ref.shape[0]
    Q = cbt_ref.shape[0]
    K = cbt_ref.shape[2]

    r = x_ref[...]
    out = jnp.zeros_like(r)
    iota_k = lax.broadcasted_iota(jnp.int32, (M, K), 1)
    lane128 = lax.broadcasted_iota(jnp.int32, (M, 128), 1)
    lane8 = lax.broadcasted_iota(jnp.int32, (8, 128), 1)
    sub8 = lax.broadcasted_iota(jnp.int32, (8, 128), 0)
    idxacc = jnp.zeros((M, 128), dtype=jnp.int32)
    lvec = jnp.zeros((8, 128), dtype=jnp.float32)

    DIM = x_ref.shape[1]
    for q in range(Q):
        e2 = jnp.dot(r * -2.0, cbt_ref[q], preferred_element_type=jnp.float32)
        rn2 = jnp.sum(r * r, axis=1, keepdims=True)
        d = (rn2 + e2) + cn2_ref[q]
        dmin = jnp.min(d, axis=1, keepdims=True)
        idx = jnp.min(jnp.where(d == dmin, iota_k, K), axis=1, keepdims=True)
        onehot = (iota_k == idx).astype(jnp.bfloat16)
        qcat = jnp.dot(onehot, split_ref[q], preferred_element_type=jnp.float32)
        quant = ((qcat[:, :DIM] + qcat[:, DIM:2 * DIM])
                 + qcat[:, 2 * DIM:])
        diff = quant - r
        lvec = lvec + jnp.where((lane8 == q) & (sub8 == 0),
                                jnp.sum(diff * diff), 0.0)
        idxacc = jnp.where(lane128 == q, jnp.broadcast_to(idx, (M, 128)), idxacc)
        out = out + (r + diff)
        r = r - quant

    out_ref[...] = out
    idx_ref[...] = idxacc
    loss_ref[0] = lvec


def _rvq_shard(x, codebooks, total_elems):
    B, N, DIM = x.shape
    Q, K, _ = codebooks.shape
    BN = B * N
    M = 1152 if BN % 1152 == 0 else BN
    G = BN // M

    xf = x.reshape(BN, DIM)
    cbt = jnp.transpose(codebooks, (0, 2, 1))
    cn2 = jnp.stack([jnp.sum(codebooks[q] * codebooks[q], axis=-1)
                     for q in range(Q)])[:, None, :]
    def _trunc_bf16(v):
        bits = lax.bitcast_convert_type(v, jnp.uint32)
        return lax.bitcast_convert_type(bits & jnp.uint32(0xFFFF0000),
                                        jnp.float32)
    hi_f = _trunc_bf16(codebooks)
    rem1 = codebooks - hi_f
    mid_f = _trunc_bf16(rem1)
    rem2 = rem1 - mid_f
    split = jnp.concatenate([hi_f.astype(jnp.bfloat16),
                             mid_f.astype(jnp.bfloat16),
                             rem2.astype(jnp.bfloat16)], axis=-1)

    out, idx_raw, loss_raw = pl.pallas_call(
        _rvq_body,
        grid=(G,),
        in_specs=[
            pl.BlockSpec((M, DIM), lambda i: (i, 0)),
            pl.BlockSpec((Q, DIM, K), lambda i: (0, 0, 0)),
            pl.BlockSpec((Q, K, 3 * DIM), lambda i: (0, 0, 0)),
            pl.BlockSpec((Q, 1, K), lambda i: (0, 0, 0)),
        ],
        out_specs=[
            pl.BlockSpec((M, DIM), lambda i: (i, 0)),
            pl.BlockSpec((M, 128), lambda i: (i, 0)),
            pl.BlockSpec((1, 8, 128), lambda i: (i, 0, 0)),
        ],
        out_shape=[
            jax.ShapeDtypeStruct((BN, DIM), jnp.float32),
            jax.ShapeDtypeStruct((BN, 128), jnp.int32),
            jax.ShapeDtypeStruct((G, 8, 128), jnp.float32),
        ],
        compiler_params=pltpu.CompilerParams(
            dimension_semantics=("arbitrary",),
        ),
    )(xf, cbt, split, cn2)

    quantized_out = out.reshape(B, N, DIM)
    indices = idx_raw[:, :Q].reshape(B, N, Q)
    losses = loss_raw.sum(axis=(0, 1))[:Q] / total_elems
    return quantized_out, indices, losses


def kernel(x, codebooks):
    B, N, DIM = x.shape
    return _rvq_shard(x, codebooks, B * N * DIM)


# speedup vs baseline: 1.3401x; 1.0283x over previous
import jax
import jax.numpy as jnp
from jax import lax
from jax.experimental import pallas as pl
from jax.experimental.pallas import tpu as pltpu


def _rvq_body(x_ref, cb_ref, hi_ref, mid_ref, lo_ref, cn2_ref,
              out_ref, idx_ref, loss_ref):
    M = x_ref.shape[0]
    Q = cb_ref.shape[0]
    K = cb_ref.shape[1]

    r = x_ref[...]
    out = jnp.zeros_like(r)
    iota_k = lax.broadcasted_iota(jnp.int32, (M, K), 1)
    lane128 = lax.broadcasted_iota(jnp.int32, (M, 128), 1)
    lane8 = lax.broadcasted_iota(jnp.int32, (8, 128), 1)
    sub8 = lax.broadcasted_iota(jnp.int32, (8, 128), 0)
    idxacc = jnp.zeros((M, 128), dtype=jnp.int32)
    lvec = jnp.zeros((8, 128), dtype=jnp.float32)

    DIM = x_ref.shape[1]
    for q in range(Q):
        e2 = lax.dot_general(r * -2.0, cb_ref[q],
                             (((1,), (1,)), ((), ())),
                             preferred_element_type=jnp.float32)
        rn2 = jnp.sum(r * r, axis=1, keepdims=True)
        d = (rn2 + e2) + cn2_ref[q]
        dmin = jnp.min(d, axis=1, keepdims=True)
        idx = jnp.min(jnp.where(d == dmin, iota_k, K), axis=1, keepdims=True)
        onehot = (iota_k == idx).astype(jnp.bfloat16)
        qhi = jnp.dot(onehot, hi_ref[q], preferred_element_type=jnp.float32)
        qmid = jnp.dot(onehot, mid_ref[q], preferred_element_type=jnp.float32)
        qlo = jnp.dot(onehot, lo_ref[q], preferred_element_type=jnp.float32)
        quant = (qhi + qmid) + qlo
        diff = quant - r
        lvec = lvec + jnp.where((lane8 == q) & (sub8 == 0),
                                jnp.sum(diff * diff), 0.0)
        idxacc = jnp.where(lane128 == q, jnp.broadcast_to(idx, (M, 128)), idxacc)
        out = out + (r + diff)
        r = r - quant

    out_ref[...] = out
    idx_ref[...] = idxacc
    loss_ref[0] = lvec


def _rvq_shard(x, codebooks, total_elems):
    B, N, DIM = x.shape
    Q, K, _ = codebooks.shape
    BN = B * N
    M = 1152 if BN % 1152 == 0 else BN
    G = BN // M

    xf = x.reshape(BN, DIM)
    cn2 = jnp.stack([jnp.sum(codebooks[q] * codebooks[q], axis=-1)
                     for q in range(Q)])[:, None, :]
    def _trunc_bf16(v):
        bits = lax.bitcast_convert_type(v, jnp.uint32)
        return lax.bitcast_convert_type(bits & jnp.uint32(0xFFFF0000),
                                        jnp.float32)
    hi_f = _trunc_bf16(codebooks)
    rem1 = codebooks - hi_f
    mid_f = _trunc_bf16(rem1)
    rem2 = rem1 - mid_f
    hi = hi_f.astype(jnp.bfloat16)
    mid = mid_f.astype(jnp.bfloat16)
    lo = rem2.astype(jnp.bfloat16)

    out, idx_raw, loss_raw = pl.pallas_call(
        _rvq_body,
        grid=(G,),
        in_specs=[
            pl.BlockSpec((M, DIM), lambda i: (i, 0)),
            pl.BlockSpec((Q, K, DIM), lambda i: (0, 0, 0)),
            pl.BlockSpec((Q, K, DIM), lambda i: (0, 0, 0)),
            pl.BlockSpec((Q, K, DIM), lambda i: (0, 0, 0)),
            pl.BlockSpec((Q, K, DIM), lambda i: (0, 0, 0)),
            pl.BlockSpec((Q, 1, K), lambda i: (0, 0, 0)),
        ],
        out_specs=[
            pl.BlockSpec((M, DIM), lambda i: (i, 0)),
            pl.BlockSpec((M, 128), lambda i: (i, 0)),
            pl.BlockSpec((1, 8, 128), lambda i: (i, 0, 0)),
        ],
        out_shape=[
            jax.ShapeDtypeStruct((BN, DIM), jnp.float32),
            jax.ShapeDtypeStruct((BN, 128), jnp.int32),
            jax.ShapeDtypeStruct((G, 8, 128), jnp.float32),
        ],
        compiler_params=pltpu.CompilerParams(
            dimension_semantics=("arbitrary",),
        ),
    )(xf, codebooks, hi, mid, lo, cn2)

    quantized_out = out.reshape(B, N, DIM)
    indices = idx_raw[:, :Q].reshape(B, N, Q)
    losses = loss_raw.sum(axis=(0, 1))[:Q] / total_elems
    return quantized_out, indices, losses


def kernel(x, codebooks):
    B, N, DIM = x.shape
    return _rvq_shard(x, codebooks, B * N * DIM)


# speedup vs baseline: 1.4032x; 1.0471x over previous
import jax
import jax.numpy as jnp
from jax import lax
from jax.experimental import pallas as pl
from jax.experimental.pallas import tpu as pltpu


def _rvq_body(x_ref, cb_ref, cn2_ref,
              out_ref, idx_ref, loss_ref,
              hi_ref, mid_ref, lo_ref):
    M = x_ref.shape[0]
    Q = cb_ref.shape[0]
    K = cb_ref.shape[1]

    @pl.when(pl.program_id(0) == 0)
    def _split():
        c = cb_ref[...]
        bits = lax.bitcast_convert_type(c, jnp.uint32)
        hi_f = lax.bitcast_convert_type(bits & jnp.uint32(0xFFFF0000),
                                        jnp.float32)
        rem1 = c - hi_f
        b2 = lax.bitcast_convert_type(rem1, jnp.uint32)
        mid_f = lax.bitcast_convert_type(b2 & jnp.uint32(0xFFFF0000),
                                         jnp.float32)
        hi_ref[...] = hi_f.astype(jnp.bfloat16)
        mid_ref[...] = mid_f.astype(jnp.bfloat16)
        lo_ref[...] = (rem1 - mid_f).astype(jnp.bfloat16)

    r = x_ref[...]
    out = jnp.zeros_like(r)
    iota_k = lax.broadcasted_iota(jnp.int32, (M, K), 1)
    lane128 = lax.broadcasted_iota(jnp.int32, (M, 128), 1)
    lane8 = lax.broadcasted_iota(jnp.int32, (8, 128), 1)
    sub8 = lax.broadcasted_iota(jnp.int32, (8, 128), 0)
    idxacc = jnp.zeros((M, 128), dtype=jnp.int32)
    lvec = jnp.zeros((8, 128), dtype=jnp.float32)

    DIM = x_ref.shape[1]
    for q in range(Q):
        e2 = lax.dot_general(r * -2.0, cb_ref[q],
                             (((1,), (1,)), ((), ())),
                             preferred_element_type=jnp.float32)
        rn2 = jnp.sum(r * r, axis=1, keepdims=True)
        d = (rn2 + e2) + cn2_ref[q]
        dmin = jnp.min(d, axis=1, keepdims=True)
        idx = jnp.min(jnp.where(d == dmin, iota_k, K), axis=1, keepdims=True)
        onehot = (iota_k == idx).astype(jnp.bfloat16)
        qhi = jnp.dot(onehot, hi_ref[q], preferred_element_type=jnp.float32)
        qmid = jnp.dot(onehot, mid_ref[q], preferred_element_type=jnp.float32)
        qlo = jnp.dot(onehot, lo_ref[q], preferred_element_type=jnp.float32)
        quant = (qhi + qmid) + qlo
        diff = quant - r
        lvec = lvec + jnp.where((lane8 == q) & (sub8 == 0),
                                jnp.sum(diff * diff), 0.0)
        idxacc = jnp.where(lane128 == q, jnp.broadcast_to(idx, (M, 128)), idxacc)
        out = out + (r + diff)
        r = r - quant

    out_ref[...] = out
    idx_ref[...] = idxacc
    loss_ref[0] = lvec


def _rvq_shard(x, codebooks, total_elems):
    B, N, DIM = x.shape
    Q, K, _ = codebooks.shape
    BN = B * N
    M = 1152 if BN % 1152 == 0 else BN
    G = BN // M

    xf = x.reshape(BN, DIM)
    cn2 = jnp.stack([jnp.sum(codebooks[q] * codebooks[q], axis=-1)
                     for q in range(Q)])[:, None, :]
    out, idx_raw, loss_raw = pl.pallas_call(
        _rvq_body,
        grid=(G,),
        in_specs=[
            pl.BlockSpec((M, DIM), lambda i: (i, 0)),
            pl.BlockSpec((Q, K, DIM), lambda i: (0, 0, 0)),
            pl.BlockSpec((Q, 1, K), lambda i: (0, 0, 0)),
        ],
        scratch_shapes=[
            pltpu.VMEM((Q, K, DIM), jnp.bfloat16),
            pltpu.VMEM((Q, K, DIM), jnp.bfloat16),
            pltpu.VMEM((Q, K, DIM), jnp.bfloat16),
        ],
        out_specs=[
            pl.BlockSpec((M, DIM), lambda i: (i, 0)),
            pl.BlockSpec((M, 128), lambda i: (i, 0)),
            pl.BlockSpec((1, 8, 128), lambda i: (i, 0, 0)),
        ],
        out_shape=[
            jax.ShapeDtypeStruct((BN, DIM), jnp.float32),
            jax.ShapeDtypeStruct((BN, 128), jnp.int32),
            jax.ShapeDtypeStruct((G, 8, 128), jnp.float32),
        ],
        compiler_params=pltpu.CompilerParams(
            dimension_semantics=("arbitrary",),
        ),
    )(xf, codebooks, cn2)

    quantized_out = out.reshape(B, N, DIM)
    indices = idx_raw[:, :Q].reshape(B, N, Q)
    losses = loss_raw.sum(axis=(0, 1))[:Q] / total_elems
    return quantized_out, indices, losses


def kernel(x, codebooks):
    B, N, DIM = x.shape
    return _rvq_shard(x, codebooks, B * N * DIM)


# speedup vs baseline: 1.4344x; 1.0222x over previous
import jax
import jax.numpy as jnp
from jax import lax
from jax.experimental import pallas as pl
from jax.experimental.pallas import tpu as pltpu


def _rvq_body(x_ref, cb_ref,
              out_ref, idx_ref, loss_ref,
              hi_ref, mid_ref, lo_ref, cn2_ref):
    M = x_ref.shape[0]
    Q = cb_ref.shape[0]
    K = cb_ref.shape[1]

    @pl.when(pl.program_id(0) == 0)
    def _split():
        c = cb_ref[...]
        bits = lax.bitcast_convert_type(c, jnp.uint32)
        hi_f = lax.bitcast_convert_type(bits & jnp.uint32(0xFFFF0000),
                                        jnp.float32)
        rem1 = c - hi_f
        b2 = lax.bitcast_convert_type(rem1, jnp.uint32)
        mid_f = lax.bitcast_convert_type(b2 & jnp.uint32(0xFFFF0000),
                                         jnp.float32)
        hi_ref[...] = hi_f.astype(jnp.bfloat16)
        mid_ref[...] = mid_f.astype(jnp.bfloat16)
        lo_ref[...] = (rem1 - mid_f).astype(jnp.bfloat16)
        cn2_ref[...] = jnp.sum(c * c, axis=-1)[:, None, :]

    r = x_ref[...]
    out = jnp.zeros_like(r)
    iota_k = lax.broadcasted_iota(jnp.int32, (M, K), 1)
    lane128 = lax.broadcasted_iota(jnp.int32, (M, 128), 1)
    lane8 = lax.broadcasted_iota(jnp.int32, (8, 128), 1)
    sub8 = lax.broadcasted_iota(jnp.int32, (8, 128), 0)
    idxacc = jnp.zeros((M, 128), dtype=jnp.int32)
    lvec = jnp.zeros((8, 128), dtype=jnp.float32)

    DIM = x_ref.shape[1]
    for q in range(Q):
        e2 = lax.dot_general(r * -2.0, cb_ref[q],
                             (((1,), (1,)), ((), ())),
                             preferred_element_type=jnp.float32)
        rn2 = jnp.sum(r * r, axis=1, keepdims=True)
        d = (rn2 + e2) + cn2_ref[q]
        dmin = jnp.min(d, axis=1, keepdims=True)
        idx = jnp.min(jnp.where(d == dmin, iota_k, K), axis=1, keepdims=True)
        onehot = (iota_k == idx).astype(jnp.bfloat16)
        qhi = jnp.dot(onehot, hi_ref[q], preferred_element_type=jnp.float32)
        qmid = jnp.dot(onehot, mid_ref[q], preferred_element_type=jnp.float32)
        qlo = jnp.dot(onehot, lo_ref[q], preferred_element_type=jnp.float32)
        quant = (qhi + qmid) + qlo
        diff = quant - r
        lvec = lvec + jnp.where((lane8 == q) & (sub8 == 0),
                                jnp.sum(diff * diff), 0.0)
        idxacc = jnp.where(lane128 == q, jnp.broadcast_to(idx, (M, 128)), idxacc)
        out = out + (r + diff)
        r = r - quant

    out_ref[...] = out
    idx_ref[...] = idxacc
    loss_ref[0] = lvec


def _rvq_shard(x, codebooks, total_elems):
    B, N, DIM = x.shape
    Q, K, _ = codebooks.shape
    BN = B * N
    M = 1152 if BN % 1152 == 0 else BN
    G = BN // M

    xf = x.reshape(BN, DIM)
    out, idx_raw, loss_raw = pl.pallas_call(
        _rvq_body,
        grid=(G,),
        in_specs=[
            pl.BlockSpec((M, DIM), lambda i: (i, 0)),
            pl.BlockSpec((Q, K, DIM), lambda i: (0, 0, 0)),
        ],
        scratch_shapes=[
            pltpu.VMEM((Q, K, DIM), jnp.bfloat16),
            pltpu.VMEM((Q, K, DIM), jnp.bfloat16),
            pltpu.VMEM((Q, K, DIM), jnp.bfloat16),
            pltpu.VMEM((Q, 1, K), jnp.float32),
        ],
        out_specs=[
            pl.BlockSpec((M, DIM), lambda i: (i, 0)),
            pl.BlockSpec((M, 128), lambda i: (i, 0)),
            pl.BlockSpec((1, 8, 128), lambda i: (i, 0, 0)),
        ],
        out_shape=[
            jax.ShapeDtypeStruct((BN, DIM), jnp.float32),
            jax.ShapeDtypeStruct((BN, 128), jnp.int32),
            jax.ShapeDtypeStruct((G, 8, 128), jnp.float32),
        ],
        compiler_params=pltpu.CompilerParams(
            dimension_semantics=("arbitrary",),
        ),
    )(xf, codebooks)

    quantized_out = out.reshape(B, N, DIM)
    indices = idx_raw[:, :Q].reshape(B, N, Q)
    losses = loss_raw.sum(axis=(0, 1))[:Q] / total_elems
    return quantized_out, indices, losses


def kernel(x, codebooks):
    B, N, DIM = x.shape
    return _rvq_shard(x, codebooks, B * N * DIM)


# speedup vs baseline: 1.5194x; 1.0593x over previous
import jax
import jax.numpy as jnp
from jax import lax
from jax.experimental import pallas as pl
from jax.experimental.pallas import tpu as pltpu


def _rvq_body(x_ref, cb_ref,
              out_ref, idx_ref, loss_ref,
              hi_ref, mid_ref, lo_ref, cn2_ref):
    M = x_ref.shape[0]
    Q = cb_ref.shape[0]
    K = cb_ref.shape[1]

    @pl.when(pl.program_id(0) == 0)
    def _split():
        c = cb_ref[...]
        bits = lax.bitcast_convert_type(c, jnp.uint32)
        hi_f = lax.bitcast_convert_type(bits & jnp.uint32(0xFFFF0000),
                                        jnp.float32)
        rem1 = c - hi_f
        b2 = lax.bitcast_convert_type(rem1, jnp.uint32)
        mid_f = lax.bitcast_convert_type(b2 & jnp.uint32(0xFFFF0000),
                                         jnp.float32)
        hi_ref[...] = hi_f.astype(jnp.bfloat16)
        mid_ref[...] = mid_f.astype(jnp.bfloat16)
        lo_ref[...] = (rem1 - mid_f).astype(jnp.bfloat16)
        cn2_ref[...] = jnp.sum(c * c, axis=-1)[:, None, :]

    r = x_ref[...]
    out = jnp.zeros_like(r)
    iota_f = lax.broadcasted_iota(jnp.int32, (M, K), 1).astype(jnp.float32)
    lane128 = lax.broadcasted_iota(jnp.int32, (M, 128), 1)
    lane8 = lax.broadcasted_iota(jnp.int32, (8, 128), 1)
    sub8 = lax.broadcasted_iota(jnp.int32, (8, 128), 0)
    idxacc = jnp.zeros((M, 128), dtype=jnp.int32)
    lvec = jnp.zeros((8, 128), dtype=jnp.float32)

    DIM = x_ref.shape[1]
    for q in range(Q):
        e2 = lax.dot_general(r * -2.0, cb_ref[q],
                             (((1,), (1,)), ((), ())),
                             preferred_element_type=jnp.float32)
        rn2 = jnp.sum(r * r, axis=1, keepdims=True)
        d = (rn2 + e2) + cn2_ref[q]
        dmin = jnp.min(d, axis=1, keepdims=True)
        idxf = jnp.min(jnp.where(d == dmin, iota_f, float(K)),
                       axis=1, keepdims=True)
        idx = idxf.astype(jnp.int32)
        onehot = (iota_f == idxf).astype(jnp.bfloat16)
        qhi = jnp.dot(onehot, hi_ref[q], preferred_element_type=jnp.float32)
        qmid = jnp.dot(onehot, mid_ref[q], preferred_element_type=jnp.float32)
        qlo = jnp.dot(onehot, lo_ref[q], preferred_element_type=jnp.float32)
        quant = (qhi + qmid) + qlo
        diff = quant - r
        lvec = lvec + jnp.where((lane8 == q) & (sub8 == 0),
                                jnp.sum(diff * diff), 0.0)
        idxacc = jnp.where(lane128 == q, jnp.broadcast_to(idx, (M, 128)), idxacc)
        out = out + (r + diff)
        r = r - quant

    out_ref[...] = out
    idx_ref[...] = idxacc
    loss_ref[0] = lvec


def _rvq_shard(x, codebooks, total_elems):
    B, N, DIM = x.shape
    Q, K, _ = codebooks.shape
    BN = B * N
    M = 1152 if BN % 1152 == 0 else BN
    G = BN // M

    xf = x.reshape(BN, DIM)
    out, idx_raw, loss_raw = pl.pallas_call(
        _rvq_body,
        grid=(G,),
        in_specs=[
            pl.BlockSpec((M, DIM), lambda i: (i, 0)),
            pl.BlockSpec((Q, K, DIM), lambda i: (0, 0, 0)),
        ],
        scratch_shapes=[
            pltpu.VMEM((Q, K, DIM), jnp.bfloat16),
            pltpu.VMEM((Q, K, DIM), jnp.bfloat16),
            pltpu.VMEM((Q, K, DIM), jnp.bfloat16),
            pltpu.VMEM((Q, 1, K), jnp.float32),
        ],
        out_specs=[
            pl.BlockSpec((M, DIM), lambda i: (i, 0)),
            pl.BlockSpec((M, 128), lambda i: (i, 0)),
            pl.BlockSpec((1, 8, 128), lambda i: (i, 0, 0)),
        ],
        out_shape=[
            jax.ShapeDtypeStruct((BN, DIM), jnp.float32),
            jax.ShapeDtypeStruct((BN, 128), jnp.int32),
            jax.ShapeDtypeStruct((G, 8, 128), jnp.float32),
        ],
        compiler_params=pltpu.CompilerParams(
            dimension_semantics=("arbitrary",),
        ),
    )(xf, codebooks)

    quantized_out = out.reshape(B, N, DIM)
    indices = idx_raw[:, :Q].reshape(B, N, Q)
    losses = loss_raw.sum(axis=(0, 1))[:Q] / total_elems
    return quantized_out, indices, losses


def kernel(x, codebooks):
    B, N, DIM = x.shape
    return _rvq_shard(x, codebooks, B * N * DIM)
